# baseline (speedup 1.0000x reference)
import jax
import jax.numpy as jnp
from jax.experimental import pallas as pl

N = 100000
BLK = 10000


def _mlp_kernel(x_ref, w1_ref, b1_ref, w2_ref, b2_ref, w3_ref, b3_ref,
                seg_ref, emb_ref, mar_ref):
    xb = x_ref[...]
    h = jnp.maximum(
        jnp.dot(xb, w1_ref[...], preferred_element_type=jnp.float32)
        + b1_ref[...], 0.0)
    g = jnp.maximum(
        jnp.dot(h, w2_ref[...], preferred_element_type=jnp.float32)
        + b2_ref[...], 0.0)
    out = (jnp.dot(g, w3_ref[...], preferred_element_type=jnp.float32)
           + b3_ref[...])
    seg_ref[...] = out[:, :5]
    emb_ref[...] = out[:, 5:8]
    mar_ref[...] = out[:, 8:9]


def kernel(x, frag_ids, group_ids, edge_index1, edge_index2, params):
    p = params
    w1 = p["Wb"]
    b1 = p["bb"].reshape(1, -1)
    w2 = jnp.concatenate([p["Ws"], p["Wi"]], axis=1)
    b2 = jnp.concatenate([p["bs"], p["bi"]]).reshape(1, -1)
    z54 = jnp.zeros((16, 4), jnp.float32)
    z55 = jnp.zeros((16, 5), jnp.float32)
    w3 = jnp.concatenate(
        [jnp.concatenate([p["Wcls"], z54], axis=1),
         jnp.concatenate([z55, p["Wemb"]], axis=1)], axis=0)
    b3 = jnp.concatenate([p["bcls"], p["bemb"]]).reshape(1, -1)

    def rows(i):
        return (i, 0)

    def whole(i):
        return (0, 0)

    seg, emb, mar = pl.pallas_call(
        _mlp_kernel,
        grid=(N // BLK,),
        in_specs=[pl.BlockSpec((BLK, 5), rows),
                  pl.BlockSpec(w1.shape, whole), pl.BlockSpec(b1.shape, whole),
                  pl.BlockSpec(w2.shape, whole), pl.BlockSpec(b2.shape, whole),
                  pl.BlockSpec(w3.shape, whole), pl.BlockSpec(b3.shape, whole)],
        out_specs=[pl.BlockSpec((BLK, 5), rows),
                   pl.BlockSpec((BLK, 3), rows),
                   pl.BlockSpec((BLK, 1), rows)],
        out_shape=[jax.ShapeDtypeStruct((N, 5), jnp.float32),
                   jax.ShapeDtypeStruct((N, 3), jnp.float32),
                   jax.ShapeDtypeStruct((N, 1), jnp.float32)],
    )(x, w1, b1, w2, b2, w3, b3)
    return (seg, emb, mar)
